# baseline (speedup 1.0000x reference)
import functools

import jax
import jax.numpy as jnp
from jax import lax
from jax.experimental import pallas as pl
from jax.experimental.pallas import tpu as pltpu

_EPS = 1e-5


def _cell_kernel(x_ref, h0_ref, w1_ref, b1_ref, g1_ref, be1_ref,
                 w2_ref, b2_ref, g2_ref, be2_ref,
                 out_ref, hlast_ref, *scratch,
                 nb, seq_len, hh, ww, cx, ch, eps):
    f_refs = scratch[:nb]
    s_refs = scratch[nb:2 * nb]
    h_refs = scratch[2 * nb:]
    hw = hh * ww
    c = cx + ch
    r0 = ww
    r1 = ww + hw
    xc, hc = c, c + cx
    t = pl.program_id(1)

    @pl.when(t == 0)
    def _init():
        for j in range(nb):
            f_refs[j][0:r0, :] = jnp.zeros_like(f_refs[j][0:r0, :])
            f_refs[j][r1:, :] = jnp.zeros_like(f_refs[j][r1:, :])
            s_refs[j][0:r0, :] = jnp.zeros_like(s_refs[j][0:r0, :])
            s_refs[j][r1:, :] = jnp.zeros_like(s_refs[j][r1:, :])
            h_refs[j][...] = h0_ref[j].reshape(hw, ch)

    pcol = lax.broadcasted_iota(jnp.int32, (hw, 1), 0) % ww
    mask_m = pcol != 0
    mask_p = pcol != ww - 1

    def shift_pair(j, col, width):
        fr, sr = f_refs[j], s_refs[j]
        fr[r0:r1, col:col + width] = jnp.where(
            mask_m, sr[r0 - 1:r1 - 1, col:col + width], 0.0
        ).astype(jnp.bfloat16)
        fr[r0:r1, 2 * c + col:2 * c + col + width] = jnp.where(
            mask_p, sr[r0 + 1:r1 + 1, col:col + width], 0.0
        ).astype(jnp.bfloat16)

    def conv3(j, w_ref, bias):
        acc = bias
        for kh in range(3):
            lhs = f_refs[j][kh * ww:kh * ww + hw, :]
            acc = acc + lax.dot_general(
                lhs, w_ref[kh], (((1,), (0,)), ((), ())),
                preferred_element_type=jnp.float32)
        return acc

    def groupnorm(y, gamma, beta):
        mu = jnp.mean(y)
        var = jnp.mean(y * y) - mu * mu
        s = lax.rsqrt(var + eps) * gamma
        return y * s + (beta - mu * s)

    hs = [h_refs[j][...] for j in range(nb)]

    for j in range(nb):
        x_bf = x_ref[j, 0].reshape(hw, cx)
        s_refs[j][r0:r1, 0:cx] = x_bf.astype(jnp.float32)
        s_refs[j][r0:r1, cx:c] = hs[j]
        f_refs[j][r0:r1, xc:xc + cx] = x_bf
        f_refs[j][r0:r1, hc:hc + ch] = hs[j].astype(jnp.bfloat16)
        shift_pair(j, 0, c)
        gates = groupnorm(conv3(j, w1_ref, b1_ref[...]),
                          g1_ref[...], be1_ref[...])
        z = jax.nn.sigmoid(gates[:, :ch])
        r = jax.nn.sigmoid(gates[:, ch:])
        rh = r * hs[j]
        s_refs[j][r0:r1, cx:c] = rh
        f_refs[j][r0:r1, hc:hc + ch] = rh.astype(jnp.bfloat16)
        shift_pair(j, cx, ch)
        cand = jnp.tanh(groupnorm(conv3(j, w2_ref, b2_ref[...]),
                                  g2_ref[...], be2_ref[...]))
        hnew = (1.0 - z) * hs[j] + z * cand
        h_refs[j][...] = hnew
        out_ref[j, 0] = hnew.reshape(hh, ww, ch).astype(out_ref.dtype)

    @pl.when(t == seq_len - 1)
    def _last():
        for j in range(nb):
            hlast_ref[j] = h_refs[j][...].reshape(hh, ww, ch).astype(
                hlast_ref.dtype)


@functools.partial(jax.jit,
                   static_argnames=("seq_len", "cin", "feat", "hh", "ww"))
def _cell_pallas(x_nhwc, h_nhwc, w1, b1, g1, be1, w2, b2, g2, be2,
                 *, seq_len, cin, feat, hh, ww):
    b = x_nhwc.shape[0]
    nb = 4 if b % 4 == 0 else (2 if b % 2 == 0 else 1)
    c = cin + feat
    kern = functools.partial(_cell_kernel, nb=nb, seq_len=seq_len, hh=hh,
                             ww=ww, cx=cin, ch=feat, eps=_EPS)
    out_shape = (
        jax.ShapeDtypeStruct((b, seq_len, hh, ww, feat), jnp.bfloat16),
        jax.ShapeDtypeStruct((b, hh, ww, feat), jnp.bfloat16),
    )
    grid_spec = pltpu.PrefetchScalarGridSpec(
        num_scalar_prefetch=0,
        grid=(b // nb, seq_len),
        in_specs=[
            pl.BlockSpec((nb, 1, hh, ww, cin), lambda i, t: (i, t, 0, 0, 0)),
            pl.BlockSpec((nb, hh, ww, feat), lambda i, t: (i, 0, 0, 0)),
            pl.BlockSpec((3, 3 * c, 2 * feat), lambda i, t: (0, 0, 0)),
            pl.BlockSpec((1, 2 * feat), lambda i, t: (0, 0)),
            pl.BlockSpec((1, 2 * feat), lambda i, t: (0, 0)),
            pl.BlockSpec((1, 2 * feat), lambda i, t: (0, 0)),
            pl.BlockSpec((3, 3 * c, feat), lambda i, t: (0, 0, 0)),
            pl.BlockSpec((1, feat), lambda i, t: (0, 0)),
            pl.BlockSpec((1, feat), lambda i, t: (0, 0)),
            pl.BlockSpec((1, feat), lambda i, t: (0, 0)),
        ],
        out_specs=(
            pl.BlockSpec((nb, 1, hh, ww, feat), lambda i, t: (i, t, 0, 0, 0)),
            pl.BlockSpec((nb, hh, ww, feat), lambda i, t: (i, 0, 0, 0)),
        ),
        scratch_shapes=(
            [pltpu.VMEM((hh * ww + 2 * ww, 3 * c), jnp.bfloat16)
             for _ in range(nb)]
            + [pltpu.VMEM((hh * ww + 2 * ww, c), jnp.float32)
               for _ in range(nb)]
            + [pltpu.VMEM((hh * ww, feat), jnp.float32) for _ in range(nb)]
        ),
    )
    return pl.pallas_call(
        kern,
        out_shape=out_shape,
        grid_spec=grid_spec,
        compiler_params=pltpu.CompilerParams(
            dimension_semantics=("parallel", "arbitrary")),
    )(x_nhwc, h_nhwc, w1, b1, g1, be1, w2, b2, g2, be2)


def kernel(w1, b1, w2, b2, gn1_g, gn1_b, gn2_g, gn2_b, inputs, h0):
    seq_len, b, cin, hh, ww = inputs.shape
    feat = h0.shape[1]
    fs = w1.shape[0]

    x_nhwc = jnp.transpose(inputs, (1, 0, 3, 4, 2)).astype(jnp.bfloat16)
    h_nhwc = jnp.transpose(h0, (0, 2, 3, 1))

    w1m = w1.reshape(fs, fs * (cin + feat), -1).astype(jnp.bfloat16)
    w2m = w2.reshape(fs, fs * (cin + feat), -1).astype(jnp.bfloat16)
    row = lambda v: v.reshape(1, -1)

    out_nhwc, hlast_nhwc = _cell_pallas(
        x_nhwc, h_nhwc,
        w1m, row(b1), row(gn1_g), row(gn1_b),
        w2m, row(b2), row(gn2_g), row(gn2_b),
        seq_len=seq_len, cin=cin, feat=feat, hh=hh, ww=ww)

    outs = jnp.transpose(out_nhwc, (1, 0, 4, 2, 3)).astype(jnp.float32)
    hlast = jnp.transpose(hlast_nhwc, (0, 3, 1, 2)).astype(jnp.float32)
    return outs, hlast


# speedup vs baseline: 1.1453x; 1.1453x over previous
import functools

import jax
import jax.numpy as jnp
from jax import lax
from jax.experimental import pallas as pl
from jax.experimental.pallas import tpu as pltpu

_EPS = 1e-5


def _cell_kernel(x_ref, h0_ref, w1_ref, b1_ref, g1_ref, be1_ref,
                 w2_ref, b2_ref, g2_ref, be2_ref,
                 out_ref, hlast_ref, *scratch,
                 nb, seq_len, hh, ww, cx, ch, eps):
    f_refs = scratch[:nb]
    s_refs = scratch[nb:2 * nb]
    h_refs = scratch[2 * nb:]
    hw = hh * ww
    c = cx + ch
    r0 = ww
    r1 = ww + hw
    xc, hc = c, c + cx
    t = pl.program_id(1)

    @pl.when(t == 0)
    def _init():
        for j in range(nb):
            f_refs[j][0:r0, :] = jnp.zeros_like(f_refs[j][0:r0, :])
            f_refs[j][r1:, :] = jnp.zeros_like(f_refs[j][r1:, :])
            s_refs[j][0:r0, :] = jnp.zeros_like(s_refs[j][0:r0, :])
            s_refs[j][r1:, :] = jnp.zeros_like(s_refs[j][r1:, :])
            h_refs[j][...] = h0_ref[j].reshape(hw, ch)

    pcol = lax.broadcasted_iota(jnp.int32, (hw, 1), 0) % ww
    mask_m = pcol != 0
    mask_p = pcol != ww - 1

    def shift_pair(j, col, width):
        fr, sr = f_refs[j], s_refs[j]
        fr[r0:r1, col:col + width] = jnp.where(
            mask_m, sr[r0 - 1:r1 - 1, col:col + width], 0.0
        ).astype(jnp.bfloat16)
        fr[r0:r1, 2 * c + col:2 * c + col + width] = jnp.where(
            mask_p, sr[r0 + 1:r1 + 1, col:col + width], 0.0
        ).astype(jnp.bfloat16)

    def conv3(j, w_ref, bias):
        acc = bias
        for kh in range(3):
            lhs = f_refs[j][kh * ww:kh * ww + hw, :]
            acc = acc + lax.dot_general(
                lhs, w_ref[kh], (((1,), (0,)), ((), ())),
                preferred_element_type=jnp.float32)
        return acc

    def groupnorm(y, gamma, beta):
        mu = jnp.mean(y)
        var = jnp.mean(y * y) - mu * mu
        s = lax.rsqrt(var + eps) * gamma
        return y * s + (beta - mu * s)

    hs = [h_refs[j][...] for j in range(nb)]

    zs, rs = [None] * nb, [None] * nb
    for j in range(nb):
        x_bf = x_ref[j, 0].reshape(hw, cx)
        s_refs[j][r0:r1, 0:cx] = x_bf.astype(jnp.float32)
        s_refs[j][r0:r1, cx:c] = hs[j]
        f_refs[j][r0:r1, xc:xc + cx] = x_bf
        f_refs[j][r0:r1, hc:hc + ch] = hs[j].astype(jnp.bfloat16)
        shift_pair(j, 0, c)
        gates = groupnorm(conv3(j, w1_ref, b1_ref[...]),
                          g1_ref[...], be1_ref[...])
        zs[j] = jax.nn.sigmoid(gates[:, :ch])
        rs[j] = jax.nn.sigmoid(gates[:, ch:])
    for j in range(nb):
        rh = rs[j] * hs[j]
        s_refs[j][r0:r1, cx:c] = rh
        f_refs[j][r0:r1, hc:hc + ch] = rh.astype(jnp.bfloat16)
        shift_pair(j, cx, ch)
        cand = jnp.tanh(groupnorm(conv3(j, w2_ref, b2_ref[...]),
                                  g2_ref[...], be2_ref[...]))
        hnew = (1.0 - zs[j]) * hs[j] + zs[j] * cand
        h_refs[j][...] = hnew
        out_ref[j, 0] = hnew.reshape(hh, ww, ch).astype(out_ref.dtype)

    @pl.when(t == seq_len - 1)
    def _last():
        for j in range(nb):
            hlast_ref[j] = h_refs[j][...].reshape(hh, ww, ch).astype(
                hlast_ref.dtype)


@functools.partial(jax.jit,
                   static_argnames=("seq_len", "cin", "feat", "hh", "ww"))
def _cell_pallas(x_nhwc, h_nhwc, w1, b1, g1, be1, w2, b2, g2, be2,
                 *, seq_len, cin, feat, hh, ww):
    b = x_nhwc.shape[0]
    nb = 4 if b % 4 == 0 else (2 if b % 2 == 0 else 1)
    c = cin + feat
    kern = functools.partial(_cell_kernel, nb=nb, seq_len=seq_len, hh=hh,
                             ww=ww, cx=cin, ch=feat, eps=_EPS)
    out_shape = (
        jax.ShapeDtypeStruct((b, seq_len, hh, ww, feat), jnp.bfloat16),
        jax.ShapeDtypeStruct((b, hh, ww, feat), jnp.bfloat16),
    )
    grid_spec = pltpu.PrefetchScalarGridSpec(
        num_scalar_prefetch=0,
        grid=(b // nb, seq_len),
        in_specs=[
            pl.BlockSpec((nb, 1, hh, ww, cin), lambda i, t: (i, t, 0, 0, 0)),
            pl.BlockSpec((nb, hh, ww, feat), lambda i, t: (i, 0, 0, 0)),
            pl.BlockSpec((3, 3 * c, 2 * feat), lambda i, t: (0, 0, 0)),
            pl.BlockSpec((1, 2 * feat), lambda i, t: (0, 0)),
            pl.BlockSpec((1, 2 * feat), lambda i, t: (0, 0)),
            pl.BlockSpec((1, 2 * feat), lambda i, t: (0, 0)),
            pl.BlockSpec((3, 3 * c, feat), lambda i, t: (0, 0, 0)),
            pl.BlockSpec((1, feat), lambda i, t: (0, 0)),
            pl.BlockSpec((1, feat), lambda i, t: (0, 0)),
            pl.BlockSpec((1, feat), lambda i, t: (0, 0)),
        ],
        out_specs=(
            pl.BlockSpec((nb, 1, hh, ww, feat), lambda i, t: (i, t, 0, 0, 0)),
            pl.BlockSpec((nb, hh, ww, feat), lambda i, t: (i, 0, 0, 0)),
        ),
        scratch_shapes=(
            [pltpu.VMEM((hh * ww + 2 * ww, 3 * c), jnp.bfloat16)
             for _ in range(nb)]
            + [pltpu.VMEM((hh * ww + 2 * ww, c), jnp.float32)
               for _ in range(nb)]
            + [pltpu.VMEM((hh * ww, feat), jnp.float32) for _ in range(nb)]
        ),
    )
    return pl.pallas_call(
        kern,
        out_shape=out_shape,
        grid_spec=grid_spec,
        compiler_params=pltpu.CompilerParams(
            dimension_semantics=("parallel", "arbitrary")),
    )(x_nhwc, h_nhwc, w1, b1, g1, be1, w2, b2, g2, be2)


def kernel(w1, b1, w2, b2, gn1_g, gn1_b, gn2_g, gn2_b, inputs, h0):
    seq_len, b, cin, hh, ww = inputs.shape
    feat = h0.shape[1]
    fs = w1.shape[0]

    x_nhwc = jnp.transpose(inputs, (1, 0, 3, 4, 2)).astype(jnp.bfloat16)
    h_nhwc = jnp.transpose(h0, (0, 2, 3, 1))

    w1m = w1.reshape(fs, fs * (cin + feat), -1).astype(jnp.bfloat16)
    w2m = w2.reshape(fs, fs * (cin + feat), -1).astype(jnp.bfloat16)
    row = lambda v: v.reshape(1, -1)

    out_nhwc, hlast_nhwc = _cell_pallas(
        x_nhwc, h_nhwc,
        w1m, row(b1), row(gn1_g), row(gn1_b),
        w2m, row(b2), row(gn2_g), row(gn2_b),
        seq_len=seq_len, cin=cin, feat=feat, hh=hh, ww=ww)

    outs = jnp.transpose(out_nhwc, (1, 0, 4, 2, 3)).astype(jnp.float32)
    hlast = jnp.transpose(hlast_nhwc, (0, 3, 1, 2)).astype(jnp.float32)
    return outs, hlast
